# Initial kernel scaffold; baseline (speedup 1.0000x reference)
import jax
import jax.numpy as jnp
from jax.experimental import pallas as pl


def kernel(vertices, edges, dofs, W0a, b0a, Wm, bm, gamma, beta):
    raise NotImplementedError("write your pallas kernel here")


# speedup vs baseline: 74.0344x; 74.0344x over previous
import functools

import jax
import jax.numpy as jnp
from jax import lax
from jax.experimental import pallas as pl
from jax.experimental.pallas import tpu as pltpu
from jax.experimental.pallas import tpu_sc as plsc

N = 10000
E = 320000
NROWS = 10240
NSUB = 16
NCORE = 2
NW = NCORE * NSUB
CHUNK = 128
NCHUNK = 79
EW = CHUNK * NCHUNK
EPAD = NW * EW
SUB_ROWS = NROWS // NSUB


def _make_sc_agg(C):
    mesh = plsc.VectorSubcoreMesh(core_axis_name="c", subcore_axis_name="s")

    @functools.partial(
        pl.kernel,
        mesh=mesh,
        out_type=jax.ShapeDtypeStruct((NCORE * NROWS, C), jnp.float32),
        scratch_types=[
            pltpu.VMEM((CHUNK,), jnp.int32),
            pltpu.VMEM((CHUNK,), jnp.int32),
            pltpu.VMEM((CHUNK, C), jnp.float32),
            pltpu.VMEM_SHARED((NROWS, C), jnp.float32),
            pltpu.SemaphoreType.DMA,
        ],
    )
    def sc_agg(x_hbm, fr_hbm, to_hbm, zero_hbm, out_hbm, toidx, fridx, rows, acc, sem):
        c = lax.axis_index("c")
        s = lax.axis_index("s")
        wid = s * jnp.int32(NCORE) + c
        r0 = s * jnp.int32(SUB_ROWS)
        pltpu.sync_copy(zero_hbm.at[pl.ds(r0, SUB_ROWS)], acc.at[pl.ds(r0, SUB_ROWS)])
        plsc.subcore_barrier()

        base = wid * jnp.int32(EW)

        def body(i, carry):
            off = base + i * jnp.int32(CHUNK)
            pltpu.sync_copy(to_hbm.at[pl.ds(off, CHUNK)], toidx)
            pltpu.sync_copy(fr_hbm.at[pl.ds(off, CHUNK)], fridx)
            pltpu.async_copy(x_hbm.at[toidx], rows, sem).wait()
            pltpu.sync_copy(rows, acc.at[fridx], add=True)
            return carry

        lax.fori_loop(jnp.int32(0), jnp.int32(NCHUNK), body, jnp.int32(0),
                      unroll=False)
        plsc.subcore_barrier()
        pltpu.sync_copy(
            acc.at[pl.ds(r0, SUB_ROWS)],
            out_hbm.at[pl.ds(c * jnp.int32(NROWS) + r0, SUB_ROWS)],
        )

    return sc_agg


_sc_agg_128 = _make_sc_agg(128)


def _make_tc_transform(delta, relu, first_prec=lax.Precision.HIGHEST):
    def body(x_ref, p_ref, d_ref, w1_ref, b1_ref, g1_ref, be1_ref,
             w2_ref, b2_ref, g2_ref, be2_ref, o_ref):
        p = p_ref[:N, :] + p_ref[NROWS:NROWS + N, :]
        x = x_ref[...]
        invd = 1.0 / d_ref[...]
        if delta:
            out = (p - p[:, 3:4] * x) * invd
        else:
            out = x + p * invd
        y = lax.dot_general(out, w1_ref[...], (((1,), (1,)), ((), ())),
                            preferred_element_type=jnp.float32,
                            precision=first_prec) + b1_ref[...]
        mu = jnp.mean(y, axis=0, keepdims=True)
        var = jnp.mean((y - mu) ** 2, axis=0, keepdims=True)
        y = (y - mu) * lax.rsqrt(var + 1e-5) * g1_ref[...] + be1_ref[...]
        z = lax.dot_general(y, w2_ref[...], (((1,), (1,)), ((), ())),
                            preferred_element_type=jnp.float32,
                            precision=lax.Precision.HIGHEST) + b2_ref[...]
        mu = jnp.mean(z, axis=0, keepdims=True)
        var = jnp.mean((z - mu) ** 2, axis=0, keepdims=True)
        z = (z - mu) * lax.rsqrt(var + 1e-5) * g2_ref[...] + be2_ref[...]
        if relu:
            z = jnp.maximum(z, 0.0)
        o_ref[...] = z

    return pl.pallas_call(
        body, out_shape=jax.ShapeDtypeStruct((N, 128), jnp.float32))


_tc_layer0 = _make_tc_transform(delta=True, relu=False,
                                first_prec=lax.Precision.DEFAULT)
_tc_layer1 = _make_tc_transform(delta=False, relu=True)
_tc_layer2 = _make_tc_transform(delta=False, relu=False)


def kernel(vertices, edges, dofs, W0a, b0a, Wm, bm, gamma, beta):
    v = vertices[0].astype(jnp.float32)
    fr = edges[0, 0, 0].astype(jnp.int32)
    to = edges[0, 0, 1].astype(jnp.int32)
    dcol = dofs[0, 0].astype(jnp.float32)[:, None]

    fr_p = jnp.concatenate([fr, jnp.full((EPAD - E,), N, jnp.int32)])
    to_p = jnp.concatenate([to, jnp.zeros((EPAD - E,), jnp.int32)])

    z128 = jnp.zeros((NROWS, 128), jnp.float32)

    def row(a):
        return a.astype(jnp.float32)[None, :]

    v128 = jnp.concatenate(
        [v, jnp.ones((N, 1), jnp.float32), jnp.zeros((N, 124), jnp.float32)], 1)
    W0a_pad = jnp.concatenate([W0a.astype(jnp.float32),
                               jnp.zeros((128, 125), jnp.float32)], 1)
    parts = _sc_agg_128(v128, fr_p, to_p, z128)
    x = _tc_layer0(v128, parts, dcol, W0a_pad, row(b0a), row(gamma[0]), row(beta[0]),
                   Wm[0].astype(jnp.float32), row(bm[0]), row(gamma[1]), row(beta[1]))

    parts = _sc_agg_128(x, fr_p, to_p, z128)
    x = _tc_layer1(x, parts, dcol, Wm[1].astype(jnp.float32), row(bm[1]),
                   row(gamma[2]), row(beta[2]), Wm[2].astype(jnp.float32),
                   row(bm[2]), row(gamma[3]), row(beta[3]))

    parts = _sc_agg_128(x, fr_p, to_p, z128)
    x = _tc_layer2(x, parts, dcol, Wm[3].astype(jnp.float32), row(bm[3]),
                   row(gamma[4]), row(beta[4]), Wm[4].astype(jnp.float32),
                   row(bm[4]), row(gamma[5]), row(beta[5]))

    return jnp.transpose(x)[None].astype(jnp.float64)
